# baseline (speedup 1.0000x reference)
import functools

import jax
import jax.numpy as jnp
from jax import lax
from jax.experimental import pallas as pl
from jax.experimental.pallas import tpu as pltpu
from jax.experimental.pallas import tpu_sc as plsc

N_NODES = 10000
N_PAD = 10240
N_EDGES = 320000
D = 128
DH = D // 2
CNTW = 16
L = 16

NC = 2
NS = 16
E_W = N_EDGES // NS
CHUNK = 40
NCH = E_W // CHUNK
NBUF = 10
ROWS_T = N_PAD // NS
WB = 160


def _sc_segment_sums(edges, recv3):
    mesh = plsc.VectorSubcoreMesh(
        core_axis_name="c", subcore_axis_name="s", num_cores=NC, num_subcores=NS
    )

    @functools.partial(
        pl.kernel,
        out_type=(
            jax.ShapeDtypeStruct((N_PAD, D), jnp.float32),
            jax.ShapeDtypeStruct((NC * N_PAD,), jnp.float32),
        ),
        mesh=mesh,
        compiler_params=pltpu.CompilerParams(
            use_tc_tiling_on_sc=False, needs_layout_passes=False
        ),
        scratch_types=(
            pltpu.VMEM((NBUF, CHUNK, DH), jnp.float32),
            pltpu.VMEM((NCH, CHUNK), jnp.int32),
            pltpu.VMEM((CHUNK, CNTW), jnp.float32),
            pltpu.VMEM((WB, DH), jnp.float32),
            pltpu.VMEM((ROWS_T, CNTW), jnp.float32),
            pltpu.VMEM((ROWS_T,), jnp.float32),
            pltpu.VMEM_SHARED((N_PAD, DH), jnp.float32),
            pltpu.VMEM_SHARED((N_PAD, CNTW), jnp.float32),
            [pltpu.SemaphoreType.DMA] * NBUF,
            [pltpu.SemaphoreType.DMA] * NBUF,
        ),
    )
    def k(edges_hbm, recv_hbm, psum_hbm, pcnt_hbm,
          ebuf, ibuf, onesv, wbuf, cbuf, cflat, acc_s, acc_c, semL, semS):
        c = lax.axis_index("c")
        s = lax.axis_index("s")
        base = s * E_W
        col = c * DH

        zval = jnp.zeros((L,), jnp.float32)
        oval = jnp.ones((L,), jnp.float32)

        def load_start(b, j):
            pltpu.async_copy(
                edges_hbm.at[pl.ds(base + j * CHUNK, CHUNK), pl.ds(col, DH)],
                ebuf.at[b], semL[b])

        def load_wait(b):
            pltpu.make_async_copy(
                edges_hbm.at[pl.ds(base, CHUNK), pl.ds(col, DH)],
                ebuf.at[b], semL[b]).wait()

        for b in range(NBUF):
            load_start(b, b)
        recv_cp = pltpu.async_copy(recv_hbm.at[s], ibuf, semS[0])

        def fill_ones(i, _):
            onesv[i, :] = oval
            return 0

        def fill_wz(i, _):
            for q in range(DH // L):
                wbuf[i, pl.ds(q * L, L)] = zval
            return 0

        def fill_cz(i, _):
            cbuf[i, :] = zval
            return 0

        lax.fori_loop(0, CHUNK, fill_ones, 0)
        lax.fori_loop(0, WB, fill_wz, 0)
        lax.fori_loop(0, ROWS_T, fill_cz, 0)

        def zbody(i, _):
            pltpu.sync_copy(wbuf, acc_s.at[pl.ds(s * ROWS_T + i * WB, WB)])
            return 0

        lax.fori_loop(0, ROWS_T // WB, zbody, 0)
        pltpu.sync_copy(cbuf, acc_c.at[pl.ds(s * ROWS_T, ROWS_T)])
        recv_cp.wait()
        plsc.subcore_barrier()

        def scat_start(b, j):
            pltpu.async_copy(ebuf.at[b], acc_s.at[ibuf.at[j]], semS[b], add=True)

            @pl.when(c == j % 2)
            def _():
                pltpu.async_copy(onesv, acc_c.at[ibuf.at[j]], semS[b], add=True)

        def scat_wait(b, j):
            pltpu.make_async_copy(ebuf.at[b], acc_s.at[ibuf.at[0]], semS[b]).wait()

            @pl.when(c == j % 2)
            def _():
                pltpu.make_async_copy(onesv, acc_c.at[ibuf.at[0]], semS[b]).wait()

        def body(g, _):
            j0 = g * NBUF
            for b in range(NBUF):
                load_wait(b)
                scat_start(b, j0 + b)
            for b in range(NBUF):
                scat_wait(b, j0 + b)

                @pl.when(g < NCH // NBUF - 1)
                def _():
                    load_start(b, j0 + NBUF + b)

            return 0

        lax.fori_loop(0, NCH // NBUF, body, 0)
        plsc.subcore_barrier()

        def wb_body(i, _):
            r = s * ROWS_T + i * WB
            pltpu.sync_copy(acc_s.at[pl.ds(r, WB)], wbuf)
            pltpu.sync_copy(wbuf, psum_hbm.at[pl.ds(r, WB), pl.ds(col, DH)])
            return 0

        lax.fori_loop(0, ROWS_T // WB, wb_body, 0)

        pltpu.sync_copy(acc_c.at[pl.ds(s * ROWS_T, ROWS_T)], cbuf)
        zidx = jnp.zeros((L,), jnp.int32)

        def gat(i, _):
            rows = i * L + lax.iota(jnp.int32, L)
            v = plsc.load_gather(cbuf, [rows, zidx])
            cflat[pl.ds(i * L, L)] = v
            return 0

        lax.fori_loop(0, ROWS_T // L, gat, 0)
        pltpu.sync_copy(cflat, pcnt_hbm.at[pl.ds(c * N_PAD + s * ROWS_T, ROWS_T)])

    return k(edges, recv3)


def _tc_combine_project(psum, pcnt, nodes, W):
    BLK = 1000
    grid = N_NODES // BLK
    cdims = (((1,), (1,)), ((), ()))

    def body(psum_ref, pcnt_ref, nodes_ref, w_ref, out_ref):
        cnt = pcnt_ref[0] + pcnt_ref[1]
        agg = psum_ref[...] / jnp.maximum(cnt, 1.0)
        acc = lax.dot_general(agg, w_ref[:, :D], cdims,
                              preferred_element_type=jnp.float32)
        acc += lax.dot_general(nodes_ref[...], w_ref[:, D:], cdims,
                               preferred_element_type=jnp.float32)
        out_ref[...] = jnp.maximum(acc, 0.0)

    return pl.pallas_call(
        body,
        grid=(grid,),
        in_specs=[
            pl.BlockSpec((BLK, D), lambda i: (i, 0)),
            pl.BlockSpec((NC, BLK, 1), lambda i: (0, i, 0)),
            pl.BlockSpec((BLK, D), lambda i: (i, 0)),
            pl.BlockSpec((D, 2 * D), lambda i: (0, 0)),
        ],
        out_specs=pl.BlockSpec((BLK, D), lambda i: (i, 0)),
        out_shape=jax.ShapeDtypeStruct((N_NODES, D), jnp.float32),
    )(psum, pcnt, nodes, W)


@jax.jit
def kernel(nodes, edges, receivers, W):
    recv3 = receivers.reshape(NS, NCH, CHUNK)
    psum, pcnt_flat = _sc_segment_sums(edges, recv3)
    pcnt = pcnt_flat.reshape(NC, N_PAD, 1)
    return _tc_combine_project(psum, pcnt, nodes, W)


# speedup vs baseline: 1.0440x; 1.0440x over previous
import functools

import jax
import jax.numpy as jnp
from jax import lax
from jax.experimental import pallas as pl
from jax.experimental.pallas import tpu as pltpu
from jax.experimental.pallas import tpu_sc as plsc

N_NODES = 10000
N_PAD = 10240
N_EDGES = 320000
D = 128
DH = D // 2
CNTW = 16
L = 16

NC = 2
NS = 16
E_W = N_EDGES // NS
CHUNK = 80
NCH = E_W // CHUNK
NBUF = 5
ROWS_T = N_PAD // NS
WB = 160


def _sc_segment_sums(edges, recv3):
    mesh = plsc.VectorSubcoreMesh(
        core_axis_name="c", subcore_axis_name="s", num_cores=NC, num_subcores=NS
    )

    @functools.partial(
        pl.kernel,
        out_type=(
            jax.ShapeDtypeStruct((N_PAD, D), jnp.float32),
            jax.ShapeDtypeStruct((NC * N_PAD,), jnp.float32),
        ),
        mesh=mesh,
        compiler_params=pltpu.CompilerParams(
            use_tc_tiling_on_sc=False, needs_layout_passes=False
        ),
        scratch_types=(
            pltpu.VMEM((NBUF, CHUNK, DH), jnp.float32),
            pltpu.VMEM((NCH, CHUNK), jnp.int32),
            pltpu.VMEM((CHUNK, CNTW), jnp.float32),
            pltpu.VMEM((WB, DH), jnp.float32),
            pltpu.VMEM((ROWS_T, CNTW), jnp.float32),
            pltpu.VMEM((ROWS_T,), jnp.float32),
            pltpu.VMEM_SHARED((N_PAD, DH), jnp.float32),
            pltpu.VMEM_SHARED((N_PAD, CNTW), jnp.float32),
            [pltpu.SemaphoreType.DMA] * NBUF,
            [pltpu.SemaphoreType.DMA] * NBUF,
        ),
    )
    def k(edges_hbm, recv_hbm, psum_hbm, pcnt_hbm,
          ebuf, ibuf, onesv, wbuf, cbuf, cflat, acc_s, acc_c, semL, semS):
        c = lax.axis_index("c")
        s = lax.axis_index("s")
        base = s * E_W
        col = c * DH

        zval = jnp.zeros((L,), jnp.float32)
        oval = jnp.ones((L,), jnp.float32)

        def load_start(b, j):
            pltpu.async_copy(
                edges_hbm.at[pl.ds(base + j * CHUNK, CHUNK), pl.ds(col, DH)],
                ebuf.at[b], semL[b])

        def load_wait(b):
            pltpu.make_async_copy(
                edges_hbm.at[pl.ds(base, CHUNK), pl.ds(col, DH)],
                ebuf.at[b], semL[b]).wait()

        for b in range(NBUF):
            load_start(b, b)
        recv_cp = pltpu.async_copy(recv_hbm.at[s], ibuf, semS[0])

        def fill_ones(i, _):
            onesv[i, :] = oval
            return 0

        def fill_wz(i, _):
            for q in range(DH // L):
                wbuf[i, pl.ds(q * L, L)] = zval
            return 0

        def fill_cz(i, _):
            cbuf[i, :] = zval
            return 0

        lax.fori_loop(0, CHUNK, fill_ones, 0)
        lax.fori_loop(0, WB, fill_wz, 0)
        lax.fori_loop(0, ROWS_T, fill_cz, 0)

        def zbody(i, _):
            pltpu.sync_copy(wbuf, acc_s.at[pl.ds(s * ROWS_T + i * WB, WB)])
            return 0

        lax.fori_loop(0, ROWS_T // WB, zbody, 0)
        pltpu.sync_copy(cbuf, acc_c.at[pl.ds(s * ROWS_T, ROWS_T)])
        recv_cp.wait()
        plsc.subcore_barrier()

        def scat_start(b, j):
            pltpu.async_copy(ebuf.at[b], acc_s.at[ibuf.at[j]], semS[b], add=True)

            @pl.when(c == j % 2)
            def _():
                pltpu.async_copy(onesv, acc_c.at[ibuf.at[j]], semS[b], add=True)

        def scat_wait(b, j):
            pltpu.make_async_copy(ebuf.at[b], acc_s.at[ibuf.at[0]], semS[b]).wait()

            @pl.when(c == j % 2)
            def _():
                pltpu.make_async_copy(onesv, acc_c.at[ibuf.at[0]], semS[b]).wait()

        def body(g, _):
            j0 = g * NBUF
            for b in range(NBUF):
                load_wait(b)
                scat_start(b, j0 + b)
            for b in range(NBUF):
                scat_wait(b, j0 + b)

                @pl.when(g < NCH // NBUF - 1)
                def _():
                    load_start(b, j0 + NBUF + b)

            return 0

        lax.fori_loop(0, NCH // NBUF, body, 0)
        plsc.subcore_barrier()

        def wb_body(i, _):
            r = s * ROWS_T + i * WB
            pltpu.sync_copy(acc_s.at[pl.ds(r, WB)], wbuf)
            pltpu.sync_copy(wbuf, psum_hbm.at[pl.ds(r, WB), pl.ds(col, DH)])
            return 0

        lax.fori_loop(0, ROWS_T // WB, wb_body, 0)

        pltpu.sync_copy(acc_c.at[pl.ds(s * ROWS_T, ROWS_T)], cbuf)
        zidx = jnp.zeros((L,), jnp.int32)

        def gat(i, _):
            rows = i * L + lax.iota(jnp.int32, L)
            v = plsc.load_gather(cbuf, [rows, zidx])
            cflat[pl.ds(i * L, L)] = v
            return 0

        lax.fori_loop(0, ROWS_T // L, gat, 0)
        pltpu.sync_copy(cflat, pcnt_hbm.at[pl.ds(c * N_PAD + s * ROWS_T, ROWS_T)])

    return k(edges, recv3)


def _tc_combine_project(psum, pcnt, nodes, W):
    BLK = 1000
    grid = N_NODES // BLK
    cdims = (((1,), (1,)), ((), ()))

    def body(psum_ref, pcnt_ref, nodes_ref, w_ref, out_ref):
        cnt = pcnt_ref[0] + pcnt_ref[1]
        agg = psum_ref[...] / jnp.maximum(cnt, 1.0)
        acc = lax.dot_general(agg, w_ref[:, :D], cdims,
                              preferred_element_type=jnp.float32)
        acc += lax.dot_general(nodes_ref[...], w_ref[:, D:], cdims,
                               preferred_element_type=jnp.float32)
        out_ref[...] = jnp.maximum(acc, 0.0)

    return pl.pallas_call(
        body,
        grid=(grid,),
        in_specs=[
            pl.BlockSpec((BLK, D), lambda i: (i, 0)),
            pl.BlockSpec((NC, BLK, 1), lambda i: (0, i, 0)),
            pl.BlockSpec((BLK, D), lambda i: (i, 0)),
            pl.BlockSpec((D, 2 * D), lambda i: (0, 0)),
        ],
        out_specs=pl.BlockSpec((BLK, D), lambda i: (i, 0)),
        out_shape=jax.ShapeDtypeStruct((N_NODES, D), jnp.float32),
    )(psum, pcnt, nodes, W)


@jax.jit
def kernel(nodes, edges, receivers, W):
    recv3 = receivers.reshape(NS, NCH, CHUNK)
    psum, pcnt_flat = _sc_segment_sums(edges, recv3)
    pcnt = pcnt_flat.reshape(NC, N_PAD, 1)
    return _tc_combine_project(psum, pcnt, nodes, W)
